# Initial kernel scaffold; baseline (speedup 1.0000x reference)
import jax
import jax.numpy as jnp
from jax.experimental import pallas as pl


def kernel(keys, values, routing_weights, key_cache, value_cache, cache_positions):
    raise NotImplementedError("write your pallas kernel here")


# speedup vs baseline: 10.3246x; 10.3246x over previous
import jax
import jax.numpy as jnp
from jax.experimental import pallas as pl
from jax.experimental.pallas import tpu as pltpu


def _retrieve_kernel(rpos_ref, rw_ref, kcache_ref, vcache_ref, out_ref,
                     kscr, vscr, sem):
    i = pl.program_id(0)

    @pl.when(i == 0)
    def _gather():
        T = kscr.shape[0]
        copies = []
        for t in range(T):
            p = rpos_ref[t]
            copies.append(pltpu.make_async_copy(
                kcache_ref.at[t, pl.ds(p, 1), :], kscr.at[pl.ds(t, 1), :], sem))
            copies.append(pltpu.make_async_copy(
                vcache_ref.at[t, pl.ds(p, 1), :], vscr.at[pl.ds(t, 1), :], sem))
        for c in copies:
            c.start()
        for c in copies:
            c.wait()

    rw = rw_ref[...]
    out_ref[0] = jnp.dot(rw, kscr[...], preferred_element_type=jnp.float32)
    out_ref[1] = jnp.dot(rw, vscr[...], preferred_element_type=jnp.float32)


def kernel(keys, values, routing_weights, key_cache, value_cache,
           cache_positions):
    T, C, H = key_cache.shape
    B, S, _ = routing_weights.shape
    BS = B * S
    BS_BLK = 512
    nblk = BS // BS_BLK

    rpos = ((cache_positions + 1) % C).astype(jnp.int32)
    rw2 = routing_weights.reshape(BS, T)

    grid_spec = pltpu.PrefetchScalarGridSpec(
        num_scalar_prefetch=1,
        grid=(nblk,),
        in_specs=[
            pl.BlockSpec((BS_BLK, T), lambda i, rpos_ref: (i, 0)),
            pl.BlockSpec(memory_space=pl.MemorySpace.ANY),
            pl.BlockSpec(memory_space=pl.MemorySpace.ANY),
        ],
        out_specs=pl.BlockSpec((2, BS_BLK, H), lambda i, rpos_ref: (0, i, 0)),
        scratch_shapes=[
            pltpu.VMEM((T, H), jnp.float32),
            pltpu.VMEM((T, H), jnp.float32),
            pltpu.SemaphoreType.DMA,
        ],
    )
    out = pl.pallas_call(
        _retrieve_kernel,
        grid_spec=grid_spec,
        out_shape=jax.ShapeDtypeStruct((2, BS, H), jnp.float32),
    )(rpos, rw2, key_cache, value_cache)
    return out.reshape(2, B, S, H)


# speedup vs baseline: 11.3878x; 1.1030x over previous
import jax
import jax.numpy as jnp
from jax.experimental import pallas as pl
from jax.experimental.pallas import tpu as pltpu


def _retrieve_kernel(rpos_ref, rw_ref, kcache_ref, vcache_ref, out_ref,
                     kscr, vscr, sem):
    i = pl.program_id(0)

    @pl.when(i == 0)
    def _gather():
        T = kscr.shape[0]
        copies = []
        for t in range(T):
            p = rpos_ref[t]
            copies.append(pltpu.make_async_copy(
                kcache_ref.at[t, pl.ds(p, 1), :], kscr.at[pl.ds(t, 1), :], sem))
            copies.append(pltpu.make_async_copy(
                vcache_ref.at[t, pl.ds(p, 1), :], vscr.at[pl.ds(t, 1), :], sem))
        for c in copies:
            c.start()
        for c in copies:
            c.wait()

    rw = rw_ref[...]
    out_ref[0] = jnp.dot(rw, kscr[...], preferred_element_type=jnp.float32)
    out_ref[1] = jnp.dot(rw, vscr[...], preferred_element_type=jnp.float32)


def kernel(keys, values, routing_weights, key_cache, value_cache,
           cache_positions):
    T, C, H = key_cache.shape
    B, S, _ = routing_weights.shape
    BS = B * S
    BS_BLK = 2048
    nblk = BS // BS_BLK

    rpos = ((cache_positions + 1) % C).astype(jnp.int32)
    rw2 = routing_weights.reshape(BS, T)

    grid_spec = pltpu.PrefetchScalarGridSpec(
        num_scalar_prefetch=1,
        grid=(nblk,),
        in_specs=[
            pl.BlockSpec((BS_BLK, T), lambda i, rpos_ref: (i, 0)),
            pl.BlockSpec(memory_space=pl.MemorySpace.ANY),
            pl.BlockSpec(memory_space=pl.MemorySpace.ANY),
        ],
        out_specs=pl.BlockSpec((2, BS_BLK, H), lambda i, rpos_ref: (0, i, 0)),
        scratch_shapes=[
            pltpu.VMEM((T, H), jnp.float32),
            pltpu.VMEM((T, H), jnp.float32),
            pltpu.SemaphoreType.DMA,
        ],
    )
    out = pl.pallas_call(
        _retrieve_kernel,
        grid_spec=grid_spec,
        out_shape=jax.ShapeDtypeStruct((2, BS, H), jnp.float32),
    )(rpos, rw2, key_cache, value_cache)
    return out.reshape(2, B, S, H)


# speedup vs baseline: 11.5146x; 1.0111x over previous
import jax
import jax.numpy as jnp
from jax.experimental import pallas as pl
from jax.experimental.pallas import tpu as pltpu


def _retrieve_kernel(rpos_ref, rw_ref, kcache_ref, vcache_ref, out_ref,
                     kscr, vscr, sem):
    i = pl.program_id(0)

    @pl.when(i == 0)
    def _gather():
        T = kscr.shape[0]
        copies = []
        for t in range(T):
            p = rpos_ref[t]
            copies.append(pltpu.make_async_copy(
                kcache_ref.at[t, pl.ds(p, 1), :], kscr.at[pl.ds(t, 1), :], sem))
            copies.append(pltpu.make_async_copy(
                vcache_ref.at[t, pl.ds(p, 1), :], vscr.at[pl.ds(t, 1), :], sem))
        for c in copies:
            c.start()
        for c in copies:
            c.wait()

    rw = rw_ref[...]
    out_ref[0] = jnp.dot(rw, kscr[...], preferred_element_type=jnp.float32)
    out_ref[1] = jnp.dot(rw, vscr[...], preferred_element_type=jnp.float32)


def kernel(keys, values, routing_weights, key_cache, value_cache,
           cache_positions):
    T, C, H = key_cache.shape
    B, S, _ = routing_weights.shape
    BS = B * S
    BS_BLK = 1024
    nblk = BS // BS_BLK

    rpos = ((cache_positions + 1) % C).astype(jnp.int32)
    rw2 = routing_weights.reshape(BS, T)

    grid_spec = pltpu.PrefetchScalarGridSpec(
        num_scalar_prefetch=1,
        grid=(nblk,),
        in_specs=[
            pl.BlockSpec((BS_BLK, T), lambda i, rpos_ref: (i, 0)),
            pl.BlockSpec(memory_space=pl.MemorySpace.ANY),
            pl.BlockSpec(memory_space=pl.MemorySpace.ANY),
        ],
        out_specs=pl.BlockSpec((2, BS_BLK, H), lambda i, rpos_ref: (0, i, 0)),
        scratch_shapes=[
            pltpu.VMEM((T, H), jnp.float32),
            pltpu.VMEM((T, H), jnp.float32),
            pltpu.SemaphoreType.DMA,
        ],
    )
    out = pl.pallas_call(
        _retrieve_kernel,
        grid_spec=grid_spec,
        out_shape=jax.ShapeDtypeStruct((2, BS, H), jnp.float32),
    )(rpos, rw2, key_cache, value_cache)
    return out.reshape(2, B, S, H)
